# baseline (speedup 1.0000x reference)
import functools

import jax
import jax.numpy as jnp
from jax import lax
from jax.experimental import pallas as pl
from jax.experimental.pallas import tpu as pltpu
from jax.experimental.pallas import tpu_sc as plsc

N = 100000
NUM_SEG = 512
D = 3
KDIM = 16
NC = 2
NS = 16
NW = NC * NS
BN = 2048
NPAD = 49 * BN
CHUNK = NPAD // NW
NVEC = CHUNK // 16
NBIN = NUM_SEG + 16
BINS_PER_W = NUM_SEG // NS


def _tc_elem_body(a_ref, c1_ref, c2_ref, t_ref, xp_ref, x_ref, oh_ref,
                  p0_ref, lc_ref, ld_ref):
    a = a_ref[0, 0]
    c1 = c1_ref[0, 0]
    c2 = c2_ref[0, 0]
    tv = t_ref[...][:, 0]
    dx = xp_ref[...] - x_ref[...]
    se = jnp.sum(dx * dx, axis=1)
    lc_ref[...] = c1 * jnp.exp(a * tv) * se
    dq = oh_ref[...] - p0_ref[...]
    se2 = jnp.sum(dq * dq, axis=1)
    ld_ref[...] = c2 * tv * se2


@jax.jit
def _tc_elem(a, c1, c2, t, xp, x, oh, p0):
    smem = pl.BlockSpec(memory_space=pltpu.SMEM)
    return pl.pallas_call(
        _tc_elem_body,
        grid=(NPAD // BN,),
        in_specs=[
            smem, smem, smem,
            pl.BlockSpec((BN, 1), lambda i: (i, 0)),
            pl.BlockSpec((BN, D), lambda i: (i, 0)),
            pl.BlockSpec((BN, D), lambda i: (i, 0)),
            pl.BlockSpec((BN, KDIM), lambda i: (i, 0)),
            pl.BlockSpec((BN, KDIM), lambda i: (i, 0)),
        ],
        out_specs=[
            pl.BlockSpec((BN,), lambda i: (i,)),
            pl.BlockSpec((BN,), lambda i: (i,)),
        ],
        out_shape=[
            jax.ShapeDtypeStruct((NPAD,), jnp.float32),
            jax.ShapeDtypeStruct((NPAD,), jnp.float32),
        ],
    )(a, c1, c2, t, xp, x, oh, p0)


def _sc_body(lc_hbm, ld_hbm, ids_hbm, out_hbm,
             lc_v, ld_v, ids_v, acc_c, acc_d, acc_n,
             res0, res1, res2, shared, rbuf):
    c = lax.axis_index("c")
    s = lax.axis_index("s")
    wid = c * NS + s
    base = wid * CHUNK

    pltpu.sync_copy(lc_hbm.at[pl.ds(base, CHUNK)], lc_v)
    pltpu.sync_copy(ld_hbm.at[pl.ds(base, CHUNK)], ld_v)
    pltpu.sync_copy(ids_hbm.at[pl.ds(base, CHUNK)], ids_v)

    zeros16 = jnp.zeros((16,), jnp.float32)
    for h in range(NBIN // 16):
        acc_c[pl.ds(h * 16, 16)] = zeros16
        acc_d[pl.ds(h * 16, 16)] = zeros16
        acc_n[pl.ds(h * 16, 16)] = zeros16

    ones16 = jnp.full((16,), 1.0, jnp.float32)

    def step(j, carry):
        ids = ids_v[pl.ds(j * 16, 16)]
        plsc.addupdate_scatter(acc_c, [ids], lc_v[pl.ds(j * 16, 16)])
        plsc.addupdate_scatter(acc_d, [ids], ld_v[pl.ds(j * 16, 16)])
        plsc.addupdate_scatter(acc_n, [ids], ones16)
        return carry

    lax.fori_loop(0, NVEC, step, 0)

    pltpu.sync_copy(acc_c.at[pl.ds(0, NUM_SEG)], shared.at[0, s, 0])
    pltpu.sync_copy(acc_d.at[pl.ds(0, NUM_SEG)], shared.at[1, s, 0])
    pltpu.sync_copy(acc_n.at[pl.ds(0, NUM_SEG)], shared.at[2, s, 0])
    plsc.subcore_barrier()

    accs = [[zeros16 for _ in range(BINS_PER_W // 16)] for _ in range(3)]
    for v in range(NS):
        b = v % 2
        for a in range(3):
            pltpu.sync_copy(
                shared.at[a, v, 0, pl.ds(s * BINS_PER_W, BINS_PER_W)],
                rbuf.at[a, b, 0])
        for a in range(3):
            for h in range(BINS_PER_W // 16):
                accs[a][h] = accs[a][h] + rbuf[a, b, 0, pl.ds(h * 16, 16)]
    for a, res in ((0, res0), (1, res1), (2, res2)):
        for h in range(BINS_PER_W // 16):
            res[pl.ds(h * 16, 16)] = accs[a][h]
    for a, res in ((0, res0), (1, res1), (2, res2)):
        pltpu.sync_copy(
            res,
            out_hbm.at[pl.ds(c * (3 * NUM_SEG) + a * NUM_SEG + s * BINS_PER_W,
                             BINS_PER_W)])


@jax.jit
def _sc_call(lc, ld, ids_pad):
    mesh = plsc.VectorSubcoreMesh(core_axis_name="c", subcore_axis_name="s")
    return pl.kernel(
        _sc_body,
        out_type=jax.ShapeDtypeStruct((NC * 3 * NUM_SEG,), jnp.float32),
        mesh=mesh,
        compiler_params=pltpu.CompilerParams(needs_layout_passes=False),
        scratch_types=[
            pltpu.VMEM((CHUNK,), jnp.float32),
            pltpu.VMEM((CHUNK,), jnp.float32),
            pltpu.VMEM((CHUNK,), jnp.int32),
            pltpu.VMEM((NBIN,), jnp.float32),
            pltpu.VMEM((NBIN,), jnp.float32),
            pltpu.VMEM((NBIN,), jnp.float32),
            pltpu.VMEM((BINS_PER_W,), jnp.float32),
            pltpu.VMEM((BINS_PER_W,), jnp.float32),
            pltpu.VMEM((BINS_PER_W,), jnp.float32),
            pltpu.VMEM_SHARED((3, NS, 1, NUM_SEG), jnp.float32),
            pltpu.VMEM((3, 2, 1, BINS_PER_W), jnp.float32),
        ],
    )(lc, ld, ids_pad)


def _tc_epilogue_body(p_ref, o_ref):
    p = p_ref[...]
    srow = p[0] + p[1]
    cnt = jnp.maximum(srow[2], 1.0)
    o_ref[0, :] = srow[0] / cnt
    o_ref[1, :] = srow[1] / cnt


@jax.jit
def _tc_epilogue(partial):
    return pl.pallas_call(
        _tc_epilogue_body,
        out_shape=jax.ShapeDtypeStruct((2, NUM_SEG), jnp.float32),
    )(partial)


def kernel(t, sigma1, x_pred, x, segment_ids, beta1, one_hot_x, p_0, K):
    ln_s = jnp.log(sigma1[0])
    a = jnp.reshape(-2.0 * ln_s, (1, 1))
    c1 = jnp.reshape(-ln_s, (1, 1))
    c2 = jnp.reshape(K * beta1[0], (1, 1))
    lc, ld = _tc_elem(a, c1, c2, t, x_pred, x, one_hot_x, p_0)
    ids_pad = jnp.pad(segment_ids.astype(jnp.int32), (0, NPAD - N),
                      constant_values=NUM_SEG)
    partial = _sc_call(lc, ld, ids_pad)
    return _tc_epilogue(partial.reshape(NC, 3, NUM_SEG))


# speedup vs baseline: 4.0301x; 4.0301x over previous
import functools

import jax
import jax.numpy as jnp
from jax import lax
from jax.experimental import pallas as pl
from jax.experimental.pallas import tpu as pltpu
from jax.experimental.pallas import tpu_sc as plsc

N = 100000
NUM_SEG = 512
D = 3
KDIM = 16
NC = 2
NS = 16
NW = NC * NS
BN = 2048
NPAD = 49 * BN
CHUNK = NPAD // NW
NVEC = CHUNK // 16
NBIN = NUM_SEG + 16
BINS_PER_W = NUM_SEG // NS


def _tc_elem_body(a_ref, c1_ref, c2_ref, t_ref, xp_ref, x_ref, oh_ref,
                  p0_ref, lc_ref, ld_ref):
    a = a_ref[0, 0]
    c1 = c1_ref[0, 0]
    c2 = c2_ref[0, 0]
    tv = t_ref[0, :]
    dx = xp_ref[...] - x_ref[...]
    se = jnp.sum(dx * dx, axis=0)
    lc_ref[...] = c1 * jnp.exp(a * tv) * se
    dq = oh_ref[...] - p0_ref[...]
    se2 = jnp.sum(dq * dq, axis=0)
    ld_ref[...] = c2 * tv * se2


@jax.jit
def _tc_elem(a, c1, c2, tT, xpT, xT, ohT, p0T):
    smem = pl.BlockSpec(memory_space=pltpu.SMEM)
    return pl.pallas_call(
        _tc_elem_body,
        grid=(NPAD // BN,),
        in_specs=[
            smem, smem, smem,
            pl.BlockSpec((1, BN), lambda i: (0, i)),
            pl.BlockSpec((D, BN), lambda i: (0, i)),
            pl.BlockSpec((D, BN), lambda i: (0, i)),
            pl.BlockSpec((KDIM, BN), lambda i: (0, i)),
            pl.BlockSpec((KDIM, BN), lambda i: (0, i)),
        ],
        out_specs=[
            pl.BlockSpec((BN,), lambda i: (i,)),
            pl.BlockSpec((BN,), lambda i: (i,)),
        ],
        out_shape=[
            jax.ShapeDtypeStruct((NPAD,), jnp.float32),
            jax.ShapeDtypeStruct((NPAD,), jnp.float32),
        ],
    )(a, c1, c2, tT, xpT, xT, ohT, p0T)


def _sc_body(lc_hbm, ld_hbm, ids_hbm, out_hbm,
             lc_v, ld_v, ids_v, acc_c, acc_d, acc_n,
             res0, res1, res2, shared, rbuf):
    c = lax.axis_index("c")
    s = lax.axis_index("s")
    wid = c * NS + s
    base = wid * CHUNK

    pltpu.sync_copy(lc_hbm.at[pl.ds(base, CHUNK)], lc_v)
    pltpu.sync_copy(ld_hbm.at[pl.ds(base, CHUNK)], ld_v)
    pltpu.sync_copy(ids_hbm.at[pl.ds(base, CHUNK)], ids_v)

    zeros16 = jnp.zeros((16,), jnp.float32)
    for h in range(NBIN // 16):
        acc_c[pl.ds(h * 16, 16)] = zeros16
        acc_d[pl.ds(h * 16, 16)] = zeros16
        acc_n[pl.ds(h * 16, 16)] = zeros16

    ones16 = jnp.full((16,), 1.0, jnp.float32)

    def step(j, carry):
        ids = ids_v[pl.ds(j * 16, 16)]
        plsc.addupdate_scatter(acc_c, [ids], lc_v[pl.ds(j * 16, 16)])
        plsc.addupdate_scatter(acc_d, [ids], ld_v[pl.ds(j * 16, 16)])
        plsc.addupdate_scatter(acc_n, [ids], ones16)
        return carry

    lax.fori_loop(0, NVEC, step, 0)

    pltpu.sync_copy(acc_c.at[pl.ds(0, NUM_SEG)], shared.at[0, s, 0])
    pltpu.sync_copy(acc_d.at[pl.ds(0, NUM_SEG)], shared.at[1, s, 0])
    pltpu.sync_copy(acc_n.at[pl.ds(0, NUM_SEG)], shared.at[2, s, 0])
    plsc.subcore_barrier()

    accs = [[zeros16 for _ in range(BINS_PER_W // 16)] for _ in range(3)]
    for v in range(NS):
        b = v % 2
        for a in range(3):
            pltpu.sync_copy(
                shared.at[a, v, 0, pl.ds(s * BINS_PER_W, BINS_PER_W)],
                rbuf.at[a, b, 0])
        for a in range(3):
            for h in range(BINS_PER_W // 16):
                accs[a][h] = accs[a][h] + rbuf[a, b, 0, pl.ds(h * 16, 16)]
    for a, res in ((0, res0), (1, res1), (2, res2)):
        for h in range(BINS_PER_W // 16):
            res[pl.ds(h * 16, 16)] = accs[a][h]
    for a, res in ((0, res0), (1, res1), (2, res2)):
        pltpu.sync_copy(
            res,
            out_hbm.at[pl.ds(c * (3 * NUM_SEG) + a * NUM_SEG + s * BINS_PER_W,
                             BINS_PER_W)])


@jax.jit
def _sc_call(lc, ld, ids_pad):
    mesh = plsc.VectorSubcoreMesh(core_axis_name="c", subcore_axis_name="s")
    return pl.kernel(
        _sc_body,
        out_type=jax.ShapeDtypeStruct((NC * 3 * NUM_SEG,), jnp.float32),
        mesh=mesh,
        compiler_params=pltpu.CompilerParams(needs_layout_passes=False),
        scratch_types=[
            pltpu.VMEM((CHUNK,), jnp.float32),
            pltpu.VMEM((CHUNK,), jnp.float32),
            pltpu.VMEM((CHUNK,), jnp.int32),
            pltpu.VMEM((NBIN,), jnp.float32),
            pltpu.VMEM((NBIN,), jnp.float32),
            pltpu.VMEM((NBIN,), jnp.float32),
            pltpu.VMEM((BINS_PER_W,), jnp.float32),
            pltpu.VMEM((BINS_PER_W,), jnp.float32),
            pltpu.VMEM((BINS_PER_W,), jnp.float32),
            pltpu.VMEM_SHARED((3, NS, 1, NUM_SEG), jnp.float32),
            pltpu.VMEM((3, 2, 1, BINS_PER_W), jnp.float32),
        ],
    )(lc, ld, ids_pad)


def _tc_epilogue_body(p_ref, o_ref):
    p = p_ref[...]
    srow = p[0] + p[1]
    cnt = jnp.maximum(srow[2], 1.0)
    o_ref[0, :] = srow[0] / cnt
    o_ref[1, :] = srow[1] / cnt


@jax.jit
def _tc_epilogue(partial):
    return pl.pallas_call(
        _tc_epilogue_body,
        out_shape=jax.ShapeDtypeStruct((2, NUM_SEG), jnp.float32),
    )(partial)


def kernel(t, sigma1, x_pred, x, segment_ids, beta1, one_hot_x, p_0, K):
    ln_s = jnp.log(sigma1[0])
    a = jnp.reshape(-2.0 * ln_s, (1, 1))
    c1 = jnp.reshape(-ln_s, (1, 1))
    c2 = jnp.reshape(K * beta1[0], (1, 1))
    lc, ld = _tc_elem(a, c1, c2, t.T, x_pred.T, x.T, one_hot_x.T, p_0.T)
    ids_pad = jnp.pad(segment_ids.astype(jnp.int32), (0, NPAD - N),
                      constant_values=NUM_SEG)
    partial = _sc_call(lc, ld, ids_pad)
    return _tc_epilogue(partial.reshape(NC, 3, NUM_SEG))


# speedup vs baseline: 5.5598x; 1.3796x over previous
import functools

import jax
import jax.numpy as jnp
from jax import lax
from jax.experimental import pallas as pl
from jax.experimental.pallas import tpu as pltpu
from jax.experimental.pallas import tpu_sc as plsc

N = 100000
NUM_SEG = 512
D = 3
KDIM = 16
NC = 2
NS = 16
NW = NC * NS
BN = 14336
NPAD = 7 * BN
CHUNK = NPAD // NW
NVEC = CHUNK // 16
NBIN = NUM_SEG + 16
BINS_PER_W = NUM_SEG // NS


def _tc_elem_body(a_ref, c1_ref, c2_ref, t_ref, xp_ref, x_ref, oh_ref,
                  p0_ref, lc_ref, ld_ref):
    a = a_ref[0, 0]
    c1 = c1_ref[0, 0]
    c2 = c2_ref[0, 0]
    tv = t_ref[0, :]
    dx = xp_ref[...] - x_ref[...]
    se = jnp.sum(dx * dx, axis=0)
    lc_ref[...] = c1 * jnp.exp(a * tv) * se
    dq = oh_ref[...] - p0_ref[...]
    se2 = jnp.sum(dq * dq, axis=0)
    ld_ref[...] = c2 * tv * se2


@jax.jit
def _tc_elem(a, c1, c2, tT, xpT, xT, ohT, p0T):
    smem = pl.BlockSpec(memory_space=pltpu.SMEM)
    return pl.pallas_call(
        _tc_elem_body,
        grid=(NPAD // BN,),
        in_specs=[
            smem, smem, smem,
            pl.BlockSpec((1, BN), lambda i: (0, i)),
            pl.BlockSpec((D, BN), lambda i: (0, i)),
            pl.BlockSpec((D, BN), lambda i: (0, i)),
            pl.BlockSpec((KDIM, BN), lambda i: (0, i)),
            pl.BlockSpec((KDIM, BN), lambda i: (0, i)),
        ],
        out_specs=[
            pl.BlockSpec((BN,), lambda i: (i,)),
            pl.BlockSpec((BN,), lambda i: (i,)),
        ],
        out_shape=[
            jax.ShapeDtypeStruct((NPAD,), jnp.float32),
            jax.ShapeDtypeStruct((NPAD,), jnp.float32),
        ],
    )(a, c1, c2, tT, xpT, xT, ohT, p0T)


def _sc_body(lc_hbm, ld_hbm, ids_hbm, out_hbm,
             lc_v, ld_v, ids_v, acc_c, acc_d, acc_n,
             res0, res1, res2, shared, rbuf):
    c = lax.axis_index("c")
    s = lax.axis_index("s")
    wid = c * NS + s
    base = wid * CHUNK

    pltpu.sync_copy(lc_hbm.at[pl.ds(base, CHUNK)], lc_v)
    pltpu.sync_copy(ld_hbm.at[pl.ds(base, CHUNK)], ld_v)
    pltpu.sync_copy(ids_hbm.at[pl.ds(base, CHUNK)], ids_v)

    zeros16 = jnp.zeros((16,), jnp.float32)
    for h in range(NBIN // 16):
        acc_c[pl.ds(h * 16, 16)] = zeros16
        acc_d[pl.ds(h * 16, 16)] = zeros16
        acc_n[pl.ds(h * 16, 16)] = zeros16

    ones16 = jnp.full((16,), 1.0, jnp.float32)

    def step(j, carry):
        ids = ids_v[pl.ds(j * 16, 16)]
        plsc.addupdate_scatter(acc_c, [ids], lc_v[pl.ds(j * 16, 16)])
        plsc.addupdate_scatter(acc_d, [ids], ld_v[pl.ds(j * 16, 16)])
        plsc.addupdate_scatter(acc_n, [ids], ones16)
        return carry

    lax.fori_loop(0, NVEC, step, 0)

    pltpu.sync_copy(acc_c.at[pl.ds(0, NUM_SEG)], shared.at[0, s, 0])
    pltpu.sync_copy(acc_d.at[pl.ds(0, NUM_SEG)], shared.at[1, s, 0])
    pltpu.sync_copy(acc_n.at[pl.ds(0, NUM_SEG)], shared.at[2, s, 0])
    plsc.subcore_barrier()

    accs = [[zeros16 for _ in range(BINS_PER_W // 16)] for _ in range(3)]
    for v in range(NS):
        b = v % 2
        for a in range(3):
            pltpu.sync_copy(
                shared.at[a, v, 0, pl.ds(s * BINS_PER_W, BINS_PER_W)],
                rbuf.at[a, b, 0])
        for a in range(3):
            for h in range(BINS_PER_W // 16):
                accs[a][h] = accs[a][h] + rbuf[a, b, 0, pl.ds(h * 16, 16)]
    for a, res in ((0, res0), (1, res1), (2, res2)):
        for h in range(BINS_PER_W // 16):
            res[pl.ds(h * 16, 16)] = accs[a][h]
    for a, res in ((0, res0), (1, res1), (2, res2)):
        pltpu.sync_copy(
            res,
            out_hbm.at[pl.ds(c * (3 * NUM_SEG) + a * NUM_SEG + s * BINS_PER_W,
                             BINS_PER_W)])


@jax.jit
def _sc_call(lc, ld, ids_pad):
    mesh = plsc.VectorSubcoreMesh(core_axis_name="c", subcore_axis_name="s")
    return pl.kernel(
        _sc_body,
        out_type=jax.ShapeDtypeStruct((NC * 3 * NUM_SEG,), jnp.float32),
        mesh=mesh,
        compiler_params=pltpu.CompilerParams(needs_layout_passes=False),
        scratch_types=[
            pltpu.VMEM((CHUNK,), jnp.float32),
            pltpu.VMEM((CHUNK,), jnp.float32),
            pltpu.VMEM((CHUNK,), jnp.int32),
            pltpu.VMEM((NBIN,), jnp.float32),
            pltpu.VMEM((NBIN,), jnp.float32),
            pltpu.VMEM((NBIN,), jnp.float32),
            pltpu.VMEM((BINS_PER_W,), jnp.float32),
            pltpu.VMEM((BINS_PER_W,), jnp.float32),
            pltpu.VMEM((BINS_PER_W,), jnp.float32),
            pltpu.VMEM_SHARED((3, NS, 1, NUM_SEG), jnp.float32),
            pltpu.VMEM((3, 2, 1, BINS_PER_W), jnp.float32),
        ],
    )(lc, ld, ids_pad)


def _tc_epilogue_body(p_ref, o_ref):
    p = p_ref[...]
    srow = p[0] + p[1]
    cnt = jnp.maximum(srow[2], 1.0)
    o_ref[0, :] = srow[0] / cnt
    o_ref[1, :] = srow[1] / cnt


@jax.jit
def _tc_epilogue(partial):
    return pl.pallas_call(
        _tc_epilogue_body,
        out_shape=jax.ShapeDtypeStruct((2, NUM_SEG), jnp.float32),
    )(partial)


def kernel(t, sigma1, x_pred, x, segment_ids, beta1, one_hot_x, p_0, K):
    ln_s = jnp.log(sigma1[0])
    a = jnp.reshape(-2.0 * ln_s, (1, 1))
    c1 = jnp.reshape(-ln_s, (1, 1))
    c2 = jnp.reshape(K * beta1[0], (1, 1))
    lc, ld = _tc_elem(a, c1, c2, t.T, x_pred.T, x.T, one_hot_x.T, p_0.T)
    ids_pad = jnp.pad(segment_ids.astype(jnp.int32), (0, NPAD - N),
                      constant_values=NUM_SEG)
    partial = _sc_call(lc, ld, ids_pad)
    return _tc_epilogue(partial.reshape(NC, 3, NUM_SEG))


# speedup vs baseline: 8.5004x; 1.5289x over previous
import functools

import jax
import jax.numpy as jnp
from jax import lax
from jax.experimental import pallas as pl
from jax.experimental.pallas import tpu as pltpu
from jax.experimental.pallas import tpu_sc as plsc

N = 100000
NUM_SEG = 512
D = 3
KDIM = 16
NC = 2
NS = 16
NW = NC * NS
BN = 14336
NPAD = 7 * BN
CHUNK = NPAD // NW
NVEC = CHUNK // 16
NBIN = NUM_SEG + 16
BINS_PER_W = NUM_SEG // NS


def _tc_elem_body(a_ref, c1_ref, c2_ref, t_ref, xp_ref, x_ref, oh_ref,
                  p0_ref, lc_ref, ld_ref):
    a = a_ref[0, 0]
    c1 = c1_ref[0, 0]
    c2 = c2_ref[0, 0]
    tv = t_ref[0, :]
    dx = xp_ref[...] - x_ref[...]
    se = jnp.sum(dx * dx, axis=0)
    lc_ref[...] = c1 * jnp.exp(a * tv) * se
    dq = oh_ref[...] - p0_ref[...]
    se2 = jnp.sum(dq * dq, axis=0)
    ld_ref[...] = c2 * tv * se2


@jax.jit
def _tc_elem(a, c1, c2, tT, xpT, xT, ohT, p0T):
    smem = pl.BlockSpec(memory_space=pltpu.SMEM)
    return pl.pallas_call(
        _tc_elem_body,
        grid=(NPAD // BN,),
        in_specs=[
            smem, smem, smem,
            pl.BlockSpec((1, BN), lambda i: (0, i)),
            pl.BlockSpec((D, BN), lambda i: (0, i)),
            pl.BlockSpec((D, BN), lambda i: (0, i)),
            pl.BlockSpec((KDIM, BN), lambda i: (0, i)),
            pl.BlockSpec((KDIM, BN), lambda i: (0, i)),
        ],
        out_specs=[
            pl.BlockSpec((BN,), lambda i: (i,)),
            pl.BlockSpec((BN,), lambda i: (i,)),
        ],
        out_shape=[
            jax.ShapeDtypeStruct((NPAD,), jnp.float32),
            jax.ShapeDtypeStruct((NPAD,), jnp.float32),
        ],
    )(a, c1, c2, tT, xpT, xT, ohT, p0T)


STRIDE = 197
NSTEP = 197
LAST_CHUNK = N - (NW - 1) * CHUNK


def _sc_body(lc_hbm, ld_hbm, ids_hbm, out_hbm,
             lc_v, ld_v, ids_v, acc_c, acc_d, acc_n,
             res0, res1, res2, shared, rbuf, sem_a, sem_b0, sem_b1):
    c = lax.axis_index("c")
    s = lax.axis_index("s")
    wid = c * NS + s
    base = wid * CHUNK

    h_lc = pltpu.async_copy(lc_hbm.at[pl.ds(base, CHUNK)], lc_v, sem_a)
    h_ld = pltpu.async_copy(ld_hbm.at[pl.ds(base, CHUNK)], ld_v, sem_a)

    @pl.when(wid < NW - 1)
    def _ids_full():
        pltpu.sync_copy(ids_hbm.at[pl.ds(base, CHUNK)], ids_v)

    @pl.when(wid == NW - 1)
    def _ids_tail():
        pltpu.sync_copy(ids_hbm.at[pl.ds(base, LAST_CHUNK)],
                        ids_v.at[pl.ds(0, LAST_CHUNK)])
        seg16 = jnp.full((16,), NUM_SEG, jnp.int32)
        for k in range((CHUNK - LAST_CHUNK) // 16):
            ids_v[pl.ds(LAST_CHUNK + k * 16, 16)] = seg16

    zeros16 = jnp.zeros((16,), jnp.float32)
    for h in range(NBIN // 16):
        acc_c[pl.ds(h * 16, 16)] = zeros16
        acc_d[pl.ds(h * 16, 16)] = zeros16
        acc_n[pl.ds(h * 16, 16)] = zeros16

    h_lc.wait()
    h_ld.wait()

    ones16 = jnp.full((16,), 1.0, jnp.float32)
    lane_base = lax.iota(jnp.int32, 16) * STRIDE
    limit16 = jnp.full((16,), CHUNK - 1, jnp.int32)

    def step(j, carry):
        idx = lane_base + j
        valid = idx < CHUNK
        idxc = jnp.minimum(idx, limit16)
        ids = plsc.load_gather(ids_v, [idxc])
        lcv = plsc.load_gather(lc_v, [idxc])
        ldv = plsc.load_gather(ld_v, [idxc])
        plsc.addupdate_scatter(acc_c, [ids], lcv, mask=valid)
        plsc.addupdate_scatter(acc_d, [ids], ldv, mask=valid)
        plsc.addupdate_scatter(acc_n, [ids], ones16, mask=valid)
        return carry

    lax.fori_loop(0, NSTEP, step, 0)

    pltpu.sync_copy(acc_c.at[pl.ds(0, NUM_SEG)], shared.at[0, s, 0])
    pltpu.sync_copy(acc_d.at[pl.ds(0, NUM_SEG)], shared.at[1, s, 0])
    pltpu.sync_copy(acc_n.at[pl.ds(0, NUM_SEG)], shared.at[2, s, 0])
    plsc.subcore_barrier()

    sems = (sem_b0, sem_b1)

    def fire(v, b):
        return [
            pltpu.async_copy(
                shared.at[a, v, 0, pl.ds(s * BINS_PER_W, BINS_PER_W)],
                rbuf.at[a, b, 0], sems[b])
            for a in range(3)
        ]

    accs = [[zeros16 for _ in range(BINS_PER_W // 16)] for _ in range(3)]
    pending = fire(0, 0)
    for v in range(NS):
        b = v % 2
        nxt = fire(v + 1, (v + 1) % 2) if v + 1 < NS else None
        for h_ in pending:
            h_.wait()
        for a in range(3):
            for h in range(BINS_PER_W // 16):
                accs[a][h] = accs[a][h] + rbuf[a, b, 0, pl.ds(h * 16, 16)]
        pending = nxt
    for a, res in ((0, res0), (1, res1), (2, res2)):
        for h in range(BINS_PER_W // 16):
            res[pl.ds(h * 16, 16)] = accs[a][h]
    for a, res in ((0, res0), (1, res1), (2, res2)):
        pltpu.sync_copy(
            res,
            out_hbm.at[pl.ds(c * (3 * NUM_SEG) + a * NUM_SEG + s * BINS_PER_W,
                             BINS_PER_W)])


@jax.jit
def _sc_call(lc, ld, ids_pad):
    mesh = plsc.VectorSubcoreMesh(core_axis_name="c", subcore_axis_name="s")
    return pl.kernel(
        _sc_body,
        out_type=jax.ShapeDtypeStruct((NC * 3 * NUM_SEG,), jnp.float32),
        mesh=mesh,
        compiler_params=pltpu.CompilerParams(needs_layout_passes=False),
        scratch_types=[
            pltpu.VMEM((CHUNK,), jnp.float32),
            pltpu.VMEM((CHUNK,), jnp.float32),
            pltpu.VMEM((CHUNK,), jnp.int32),
            pltpu.VMEM((NBIN,), jnp.float32),
            pltpu.VMEM((NBIN,), jnp.float32),
            pltpu.VMEM((NBIN,), jnp.float32),
            pltpu.VMEM((BINS_PER_W,), jnp.float32),
            pltpu.VMEM((BINS_PER_W,), jnp.float32),
            pltpu.VMEM((BINS_PER_W,), jnp.float32),
            pltpu.VMEM_SHARED((3, NS, 1, NUM_SEG), jnp.float32),
            pltpu.VMEM((3, 2, 1, BINS_PER_W), jnp.float32),
            pltpu.SemaphoreType.DMA,
            pltpu.SemaphoreType.DMA,
            pltpu.SemaphoreType.DMA,
        ],
    )(lc, ld, ids_pad)


def _tc_epilogue_body(p_ref, o_ref):
    s0 = p_ref[pl.ds(0, NUM_SEG)] + p_ref[pl.ds(3 * NUM_SEG, NUM_SEG)]
    s1 = p_ref[pl.ds(NUM_SEG, NUM_SEG)] + p_ref[pl.ds(4 * NUM_SEG, NUM_SEG)]
    s2 = p_ref[pl.ds(2 * NUM_SEG, NUM_SEG)] + p_ref[pl.ds(5 * NUM_SEG, NUM_SEG)]
    cnt = jnp.maximum(s2, 1.0)
    o_ref[...] = jnp.stack([s0 / cnt, s1 / cnt])


@jax.jit
def _tc_epilogue(partial):
    return pl.pallas_call(
        _tc_epilogue_body,
        out_shape=jax.ShapeDtypeStruct((2, NUM_SEG), jnp.float32),
    )(partial)


def kernel(t, sigma1, x_pred, x, segment_ids, beta1, one_hot_x, p_0, K):
    ln_s = jnp.log(sigma1[0])
    a = jnp.reshape(-2.0 * ln_s, (1, 1))
    c1 = jnp.reshape(-ln_s, (1, 1))
    c2 = jnp.reshape(K * beta1[0], (1, 1))
    lc, ld = _tc_elem(a, c1, c2, t.T, x_pred.T, x.T, one_hot_x.T, p_0.T)
    partial = _sc_call(lc, ld, segment_ids.astype(jnp.int32))
    return _tc_epilogue(partial)
